# Initial kernel scaffold; baseline (speedup 1.0000x reference)
import jax
import jax.numpy as jnp
from jax.experimental import pallas as pl


def kernel(batch_BOW, batch_indices, exp_m, exp_n, exp_s, seeds_topic_matrix, pi):
    raise NotImplementedError("write your pallas kernel here")


# speedup vs baseline: 2.2187x; 2.2187x over previous
import functools

import jax
import jax.numpy as jnp
from jax import lax
from jax.experimental import pallas as pl
from jax.experimental.pallas import tpu as pltpu

D = 100000
V = 2000
K = 64
B = 128
_beta = 0.05
_mu = 0.05
_eta = 0.1
_eps = 1e-06
_rho = 1.0 / (1 + 5) ** 0.9
BD = 4000
_F32 = jnp.float32
_PREC = lax.Precision.HIGHEST


def _body(idx_sref, exp_m_any, exp_m_blk, idx_vref, bow_ref, en_ref, es_ref,
          sd_ref, pi_ref, out_ref, gath, delta, sem):
    i = pl.program_id(0)

    @pl.when(i == 0)
    def _dense():
        def _start(j, _):
            pltpu.make_async_copy(
                exp_m_any.at[pl.ds(idx_sref[j], 1)],
                gath.at[pl.ds(j, 1)], sem).start()
            return 0

        def _wait(j, _):
            pltpu.make_async_copy(
                exp_m_any.at[pl.ds(idx_sref[j], 1)],
                gath.at[pl.ds(j, 1)], sem).wait()
            return 0

        lax.fori_loop(0, B, _start, 0)
        lax.fori_loop(0, B, _wait, 0)

        bow = bow_ref[...].astype(_F32)
        en = en_ref[...]
        es = es_ref[...]
        sd = sd_ref[...]
        pi = pi_ref[...]
        en_sum = jnp.sum(en, axis=0, keepdims=True)
        es_sum = jnp.sum(es, axis=0, keepdims=True)
        s_cnt = jnp.sum(sd, axis=0, keepdims=True)
        rate_s = (_mu + es) / (_mu * s_cnt + es_sum)
        rate_n = (_beta + en) / (_beta * V + en_sum)
        is_seed = (jnp.sum(sd, axis=1, keepdims=True) > 0).astype(_F32)
        r1 = sd * (pi * rate_s + (1.0 - pi) * rate_n)
        cm = (1.0 - sd) * rate_n
        p = sd * (pi * pi * rate_s + (1.0 - pi) * (1.0 - pi) * rate_n)
        q = (1.0 - is_seed * pi) * cm
        emb = gath[...]
        m_eta = emb + _eta
        s1 = lax.dot_general(m_eta, r1, (((1,), (1,)), ((), ())),
                             precision=_PREC, preferred_element_type=_F32)
        s2 = lax.dot_general(m_eta, cm, (((1,), (1,)), ((), ())),
                             precision=_PREC, preferred_element_type=_F32)
        u1 = bow / (s1 + _eps)
        u2 = bow / (s2 + _eps)
        t = (lax.dot_general(u1, p, (((1,), (0,)), ((), ())),
                             precision=_PREC, preferred_element_type=_F32)
             + lax.dot_general(u2, q, (((1,), (0,)), ((), ())),
                               precision=_PREC, preferred_element_type=_F32))
        new_rows = (1.0 - _rho) * emb + _rho * (m_eta * t)
        delta[...] = new_rows - emb

    rows = lax.broadcasted_iota(jnp.int32, (BD, B), 0) + i * BD
    one_hot = (rows == idx_vref[...]).astype(_F32)
    corr = lax.dot_general(one_hot, delta[...], (((1,), (0,)), ((), ())),
                           precision=_PREC, preferred_element_type=_F32)
    out_ref[...] = exp_m_blk[...] + corr


@functools.partial(jax.jit, static_argnames=("interpret",))
def kernel(batch_BOW, batch_indices, exp_m, exp_n, exp_s, seeds_topic_matrix,
           pi, interpret=False):
    grid_spec = pltpu.PrefetchScalarGridSpec(
        num_scalar_prefetch=1,
        grid=(D // BD,),
        in_specs=[
            pl.BlockSpec(memory_space=pl.ANY),
            pl.BlockSpec((BD, K), lambda i, idx: (i, 0)),
            pl.BlockSpec((1, B), lambda i, idx: (0, 0)),
            pl.BlockSpec((B, V), lambda i, idx: (0, 0)),
            pl.BlockSpec((V, K), lambda i, idx: (0, 0)),
            pl.BlockSpec((V, K), lambda i, idx: (0, 0)),
            pl.BlockSpec((V, K), lambda i, idx: (0, 0)),
            pl.BlockSpec((1, K), lambda i, idx: (0, 0)),
        ],
        out_specs=pl.BlockSpec((BD, K), lambda i, idx: (i, 0)),
        scratch_shapes=[
            pltpu.VMEM((B, K), _F32),
            pltpu.VMEM((B, K), _F32),
            pltpu.SemaphoreType.DMA,
        ],
    )
    return pl.pallas_call(
        _body,
        grid_spec=grid_spec,
        out_shape=jax.ShapeDtypeStruct((D, K), _F32),
        interpret=interpret,
    )(batch_indices, exp_m, exp_m, batch_indices.reshape(1, B), batch_BOW,
      exp_n, exp_s, seeds_topic_matrix, pi.reshape(1, K))


# speedup vs baseline: 2.5009x; 1.1272x over previous
import functools

import jax
import jax.numpy as jnp
from jax import lax
from jax.experimental import pallas as pl
from jax.experimental.pallas import tpu as pltpu

D = 100000
V = 2000
K = 64
B = 128
_beta = 0.05
_mu = 0.05
_eta = 0.1
_eps = 1e-06
_rho = 1.0 / (1 + 5) ** 0.9
BD = 4000
_F32 = jnp.float32
_PREC = lax.Precision.HIGHEST


def _copy_dense_body(idx_sref, exp_m_any, exp_m_blk, bow_ref, en_ref, es_ref,
                     sd_ref, pi_ref, out_ref, rows_ref, gath, sem):
    i = pl.program_id(0)

    @pl.when(i == 0)
    def _dense():
        def _start(j, _):
            pltpu.make_async_copy(
                exp_m_any.at[pl.ds(idx_sref[j], 1)],
                gath.at[pl.ds(j, 1)], sem).start()
            return 0

        def _wait(j, _):
            pltpu.make_async_copy(
                exp_m_any.at[pl.ds(idx_sref[j], 1)],
                gath.at[pl.ds(j, 1)], sem).wait()
            return 0

        lax.fori_loop(0, B, _start, 0)
        lax.fori_loop(0, B, _wait, 0)

        bow = bow_ref[...].astype(_F32)
        en = en_ref[...]
        es = es_ref[...]
        sd = sd_ref[...]
        pi = pi_ref[...]
        en_sum = jnp.sum(en, axis=0, keepdims=True)
        es_sum = jnp.sum(es, axis=0, keepdims=True)
        s_cnt = jnp.sum(sd, axis=0, keepdims=True)
        rate_s = (_mu + es) / (_mu * s_cnt + es_sum)
        rate_n = (_beta + en) / (_beta * V + en_sum)
        is_seed = (jnp.sum(sd, axis=1, keepdims=True) > 0).astype(_F32)
        r1 = sd * (pi * rate_s + (1.0 - pi) * rate_n)
        cm = (1.0 - sd) * rate_n
        p = sd * (pi * pi * rate_s + (1.0 - pi) * (1.0 - pi) * rate_n)
        q = (1.0 - is_seed * pi) * cm
        emb = gath[...]
        m_eta = emb + _eta
        s1 = lax.dot_general(m_eta, r1, (((1,), (1,)), ((), ())),
                             precision=_PREC, preferred_element_type=_F32)
        s2 = lax.dot_general(m_eta, cm, (((1,), (1,)), ((), ())),
                             precision=_PREC, preferred_element_type=_F32)
        u1 = bow / (s1 + _eps)
        u2 = bow / (s2 + _eps)
        t = (lax.dot_general(u1, p, (((1,), (0,)), ((), ())),
                             precision=_PREC, preferred_element_type=_F32)
             + lax.dot_general(u2, q, (((1,), (0,)), ((), ())),
                               precision=_PREC, preferred_element_type=_F32))
        rows_ref[...] = (1.0 - _rho) * emb + _rho * (m_eta * t)

    out_ref[...] = exp_m_blk[...]


def _scatter_body(idx_sref, in_any, rows_ref, out_any, sem):
    del in_any

    def _start(j, _):
        pltpu.make_async_copy(
            rows_ref.at[pl.ds(j, 1)],
            out_any.at[pl.ds(idx_sref[j], 1)], sem).start()
        return 0

    def _wait(j, _):
        pltpu.make_async_copy(
            rows_ref.at[pl.ds(j, 1)],
            out_any.at[pl.ds(idx_sref[j], 1)], sem).wait()
        return 0

    lax.fori_loop(0, B, _start, 0)
    lax.fori_loop(0, B, _wait, 0)


@functools.partial(jax.jit, static_argnames=("interpret",))
def kernel(batch_BOW, batch_indices, exp_m, exp_n, exp_s, seeds_topic_matrix,
           pi, interpret=False):
    copy_spec = pltpu.PrefetchScalarGridSpec(
        num_scalar_prefetch=1,
        grid=(D // BD,),
        in_specs=[
            pl.BlockSpec(memory_space=pl.ANY),
            pl.BlockSpec((BD, K), lambda i, idx: (i, 0)),
            pl.BlockSpec((B, V), lambda i, idx: (0, 0)),
            pl.BlockSpec((V, K), lambda i, idx: (0, 0)),
            pl.BlockSpec((V, K), lambda i, idx: (0, 0)),
            pl.BlockSpec((V, K), lambda i, idx: (0, 0)),
            pl.BlockSpec((1, K), lambda i, idx: (0, 0)),
        ],
        out_specs=[
            pl.BlockSpec((BD, K), lambda i, idx: (i, 0)),
            pl.BlockSpec((B, K), lambda i, idx: (0, 0)),
        ],
        scratch_shapes=[
            pltpu.VMEM((B, K), _F32),
            pltpu.SemaphoreType.DMA,
        ],
    )
    out0, new_rows = pl.pallas_call(
        _copy_dense_body,
        grid_spec=copy_spec,
        out_shape=[jax.ShapeDtypeStruct((D, K), _F32),
                   jax.ShapeDtypeStruct((B, K), _F32)],
        interpret=interpret,
    )(batch_indices, exp_m, exp_m, batch_BOW, exp_n, exp_s,
      seeds_topic_matrix, pi.reshape(1, K))

    scatter_spec = pltpu.PrefetchScalarGridSpec(
        num_scalar_prefetch=1,
        grid=(1,),
        in_specs=[
            pl.BlockSpec(memory_space=pl.ANY),
            pl.BlockSpec((B, K), lambda i, idx: (0, 0)),
        ],
        out_specs=pl.BlockSpec(memory_space=pl.ANY),
        scratch_shapes=[pltpu.SemaphoreType.DMA],
    )
    return pl.pallas_call(
        _scatter_body,
        grid_spec=scatter_spec,
        out_shape=jax.ShapeDtypeStruct((D, K), _F32),
        input_output_aliases={1: 0},
        interpret=interpret,
    )(batch_indices, out0, new_rows)
